# baseline (speedup 1.0000x reference)
import functools
import jax
import jax.numpy as jnp
from jax import lax
from jax.experimental import pallas as pl
from jax.experimental.pallas import tpu as pltpu
from jax.experimental.pallas import tpu_sc as plsc

BATCH = 16384
EMB = 32
PACK = 4
WIDE = PACK * EMB
NUM_CORES = 2
NUM_SUBCORES = 16
NUM_WORKERS = NUM_CORES * NUM_SUBCORES
BPW = BATCH // NUM_WORKERS
CH = 256
NCH = BPW // CH
IDXV = 128


def _mf_body(ut_hbm, uq_hbm, vt_hbm, vq_hbm, ue_hbm, ie_hbm, out_hbm,
             utile, uquar, vtile, vquar, ubuf, vbuf, outb, sem):
    wid = lax.axis_index("s") * NUM_CORES + lax.axis_index("c")
    base = wid * BPW

    pltpu.sync_copy(ut_hbm.at[wid], utile)
    pltpu.sync_copy(uq_hbm.at[wid], uquar)
    pltpu.sync_copy(vt_hbm.at[wid], vtile)
    pltpu.sync_copy(vq_hbm.at[wid], vquar)

    riota = lax.iota(jnp.int32, 16)

    def chunk(h):
        copies = []
        for p in range(CH // IDXV):
            off = h * CH + p * IDXV
            copies.append(pltpu.async_copy(
                ue_hbm.at[utile.at[pl.ds(off, IDXV)]],
                ubuf.at[pl.ds(p * IDXV, IDXV)], sem))
            copies.append(pltpu.async_copy(
                ie_hbm.at[vtile.at[pl.ds(off, IDXV)]],
                vbuf.at[pl.ds(p * IDXV, IDXV)], sem))
        for cp in copies:
            cp.wait()

        for i in range(CH // 16):
            off = h * CH + i * 16
            jvec = i * 16 + riota
            uq16 = uquar[pl.ds(off, 16)]
            vq16 = vquar[pl.ds(off, 16)]
            acc = jnp.zeros((16,), jnp.float32)
            for c in range(EMB):
                uc = plsc.load_gather(ubuf, [jvec, uq16 + c])
                vc = plsc.load_gather(vbuf, [jvec, vq16 + c])
                acc = acc + uc * vc
            outb[pl.ds(off, 16)] = acc

    for h in range(NCH):
        chunk(h)

    pltpu.sync_copy(outb, out_hbm.at[pl.ds(base, BPW)])


_mf_kernel = functools.partial(
    pl.kernel,
    mesh=plsc.VectorSubcoreMesh(core_axis_name="c", subcore_axis_name="s"),
    out_type=jax.ShapeDtypeStruct((BATCH,), jnp.float32),
    scratch_types=[
        pltpu.VMEM((BPW,), jnp.int32),
        pltpu.VMEM((BPW,), jnp.int32),
        pltpu.VMEM((BPW,), jnp.int32),
        pltpu.VMEM((BPW,), jnp.int32),
        pltpu.VMEM((CH, WIDE), jnp.float32),
        pltpu.VMEM((CH, WIDE), jnp.float32),
        pltpu.VMEM((BPW,), jnp.float32),
        pltpu.SemaphoreType.DMA,
    ],
    compiler_params=pltpu.CompilerParams(needs_layout_passes=False),
)(_mf_body)


NUM_ROWS = 1000000
QROWS = NUM_ROWS // PACK
BLK = 400
QBLKS = QROWS // BLK


def _repack_body(q0, q1, q2, q3, dst):
    dst[...] = jnp.concatenate(
        [q0[...], q1[...], q2[...], q3[...]], axis=1)


def _qspec(s):
    return pl.BlockSpec((BLK, EMB), lambda i, s=s: (s * QBLKS + i, 0))


_repack = pl.pallas_call(
    _repack_body,
    grid=(QBLKS,),
    in_specs=[_qspec(0), _qspec(1), _qspec(2), _qspec(3)],
    out_specs=pl.BlockSpec((BLK, WIDE), lambda i: (i, 0)),
    out_shape=jax.ShapeDtypeStruct((QROWS, WIDE), jnp.float32),
)


@jax.jit
def kernel(u, v, user_emb, item_emb):
    u32 = u.astype(jnp.int32)
    v32 = v.astype(jnp.int32)
    ut = (u32 % QROWS).reshape(NUM_WORKERS, BPW)
    uq = ((u32 // QROWS) * EMB).reshape(NUM_WORKERS, BPW)
    vt = (v32 % QROWS).reshape(NUM_WORKERS, BPW)
    vq = ((v32 // QROWS) * EMB).reshape(NUM_WORKERS, BPW)
    ue2 = _repack(user_emb, user_emb, user_emb, user_emb)
    ie2 = _repack(item_emb, item_emb, item_emb, item_emb)
    return _mf_kernel(ut, uq, vt, vq, ue2, ie2)


# speedup vs baseline: 1.4793x; 1.4793x over previous
import functools
import jax
import jax.numpy as jnp
from jax import lax
from jax.experimental import pallas as pl
from jax.experimental.pallas import tpu as pltpu
from jax.experimental.pallas import tpu_sc as plsc

BATCH = 16384
EMB = 32
PACK = 4
WIDE = PACK * EMB
NUM_CORES = 2
NUM_SUBCORES = 16
NUM_WORKERS = NUM_CORES * NUM_SUBCORES
BPW = BATCH // NUM_WORKERS
CH = 256
NCH = BPW // CH
IDXV = 128


def _mf_body(ut_hbm, uq_hbm, vt_hbm, vq_hbm, ue_hbm, ie_hbm, out_hbm,
             utile, uquar, vtile, vquar, ubuf, vbuf, outb, sem):
    wid = lax.axis_index("s") * NUM_CORES + lax.axis_index("c")
    base = wid * BPW

    pltpu.sync_copy(ut_hbm.at[wid], utile)
    pltpu.sync_copy(uq_hbm.at[wid], uquar)
    pltpu.sync_copy(vt_hbm.at[wid], vtile)
    pltpu.sync_copy(vq_hbm.at[wid], vquar)

    riota = lax.iota(jnp.int32, 16)

    def chunk(h):
        copies = []
        for p in range(CH // IDXV):
            off = h * CH + p * IDXV
            copies.append(pltpu.async_copy(
                ue_hbm.at[utile.at[pl.ds(off, IDXV)]],
                ubuf.at[pl.ds(p * IDXV, IDXV)], sem))
            copies.append(pltpu.async_copy(
                ie_hbm.at[vtile.at[pl.ds(off, IDXV)]],
                vbuf.at[pl.ds(p * IDXV, IDXV)], sem))
        for cp in copies:
            cp.wait()

        for i in range(CH // 16):
            off = h * CH + i * 16
            jvec = i * 16 + riota
            uq16 = uquar[pl.ds(off, 16)]
            vq16 = vquar[pl.ds(off, 16)]
            acc = jnp.zeros((16,), jnp.float32)
            for c in range(EMB):
                uc = plsc.load_gather(ubuf, [jvec, uq16 + c])
                vc = plsc.load_gather(vbuf, [jvec, vq16 + c])
                acc = acc + uc * vc
            outb[pl.ds(off, 16)] = acc

    for h in range(NCH):
        chunk(h)

    pltpu.sync_copy(outb, out_hbm.at[pl.ds(base, BPW)])


_mf_kernel = functools.partial(
    pl.kernel,
    mesh=plsc.VectorSubcoreMesh(core_axis_name="c", subcore_axis_name="s"),
    out_type=jax.ShapeDtypeStruct((BATCH,), jnp.float32),
    scratch_types=[
        pltpu.VMEM((BPW,), jnp.int32),
        pltpu.VMEM((BPW,), jnp.int32),
        pltpu.VMEM((BPW,), jnp.int32),
        pltpu.VMEM((BPW,), jnp.int32),
        pltpu.VMEM((CH, WIDE), jnp.float32),
        pltpu.VMEM((CH, WIDE), jnp.float32),
        pltpu.VMEM((BPW,), jnp.float32),
        pltpu.SemaphoreType.DMA,
    ],
    compiler_params=pltpu.CompilerParams(needs_layout_passes=False),
)(_mf_body)


NUM_ROWS = 1000000
QROWS = NUM_ROWS // PACK
BLK = 2000
QBLKS = QROWS // BLK


def _repack_body(q0, q1, q2, q3, dst):
    dst[...] = jnp.concatenate(
        [q0[...], q1[...], q2[...], q3[...]], axis=1)


def _qspec(s):
    return pl.BlockSpec((BLK, EMB), lambda i, s=s: (s * QBLKS + i, 0))


_repack = pl.pallas_call(
    _repack_body,
    grid=(QBLKS,),
    in_specs=[_qspec(0), _qspec(1), _qspec(2), _qspec(3)],
    out_specs=pl.BlockSpec((BLK, WIDE), lambda i: (i, 0)),
    out_shape=jax.ShapeDtypeStruct((QROWS, WIDE), jnp.float32),
)


@jax.jit
def kernel(u, v, user_emb, item_emb):
    u32 = u.astype(jnp.int32)
    v32 = v.astype(jnp.int32)
    ut = (u32 % QROWS).reshape(NUM_WORKERS, BPW)
    uq = ((u32 // QROWS) * EMB).reshape(NUM_WORKERS, BPW)
    vt = (v32 % QROWS).reshape(NUM_WORKERS, BPW)
    vq = ((v32 // QROWS) * EMB).reshape(NUM_WORKERS, BPW)
    ue2 = _repack(user_emb, user_emb, user_emb, user_emb)
    ie2 = _repack(item_emb, item_emb, item_emb, item_emb)
    return _mf_kernel(ut, uq, vt, vq, ue2, ie2)


# speedup vs baseline: 2.5762x; 1.7415x over previous
import functools
import jax
import jax.numpy as jnp
from jax import lax
from jax.experimental import pallas as pl
from jax.experimental.pallas import tpu as pltpu
from jax.experimental.pallas import tpu_sc as plsc

BATCH = 16384
EMB = 32
NUM_CORES = 2
NUM_SUBCORES = 16
NUM_WORKERS = NUM_CORES * NUM_SUBCORES
BPW = BATCH // NUM_WORKERS
CH = 256
NCH = BPW // CH


def _mf_body(u_hbm, v_hbm, ue_hbm, ie_hbm, out_hbm,
             uidx, vidx, spu, spv, usm, vsm, urows, vrows, outb, sem):
    sid = lax.axis_index("s")
    wid = sid * NUM_CORES + lax.axis_index("c")
    base = wid * BPW

    pltpu.sync_copy(u_hbm.at[wid], uidx)
    pltpu.sync_copy(v_hbm.at[wid], vidx)
    pltpu.sync_copy(u_hbm.at[wid], spu.at[sid])
    pltpu.sync_copy(v_hbm.at[wid], spv.at[sid])
    pltpu.sync_copy(spu.at[sid], usm)
    pltpu.sync_copy(spv.at[sid], vsm)

    riota = lax.iota(jnp.int32, 16)

    def chunk(h, carry):
        def fetch(i, c2):
            for j in range(4):
                r = i * 4 + j
                g = h * CH + r
                pltpu.async_copy(ue_hbm.at[pl.ds(usm[g], 1)],
                                 urows.at[pl.ds(r, 1)], sem)
                pltpu.async_copy(ie_hbm.at[pl.ds(vsm[g], 1)],
                                 vrows.at[pl.ds(r, 1)], sem)
            return c2

        lax.fori_loop(0, CH // 4, fetch, 0)

        pltpu.make_async_copy(ue_hbm.at[pl.ds(0, CH)], urows, sem).wait()
        pltpu.make_async_copy(ie_hbm.at[pl.ds(0, CH)], vrows, sem).wait()

        def body(i, c2):
            rows16 = i * 16 + riota
            acc = jnp.zeros((16,), jnp.float32)
            for c in range(EMB):
                cvec = jnp.full((16,), c, jnp.int32)
                uc = plsc.load_gather(urows, [rows16, cvec])
                vc = plsc.load_gather(vrows, [rows16, cvec])
                acc = acc + uc * vc
            outb[pl.ds(pl.multiple_of(h * CH + i * 16, 16), 16)] = acc
            return c2

        lax.fori_loop(0, CH // 16, body, 0)
        return carry

    lax.fori_loop(0, NCH, chunk, 0)

    pltpu.sync_copy(outb, out_hbm.at[pl.ds(base, BPW)])


_mf_kernel = functools.partial(
    pl.kernel,
    mesh=plsc.VectorSubcoreMesh(core_axis_name="c", subcore_axis_name="s"),
    out_type=jax.ShapeDtypeStruct((BATCH,), jnp.float32),
    scratch_types=[
        pltpu.VMEM((BPW,), jnp.int32),
        pltpu.VMEM((BPW,), jnp.int32),
        pltpu.VMEM_SHARED((NUM_SUBCORES, BPW), jnp.int32),
        pltpu.VMEM_SHARED((NUM_SUBCORES, BPW), jnp.int32),
        pltpu.SMEM((BPW,), jnp.int32),
        pltpu.SMEM((BPW,), jnp.int32),
        pltpu.VMEM((CH, EMB), jnp.float32),
        pltpu.VMEM((CH, EMB), jnp.float32),
        pltpu.VMEM((BPW,), jnp.float32),
        pltpu.SemaphoreType.DMA,
    ],
    compiler_params=pltpu.CompilerParams(needs_layout_passes=False),
)(_mf_body)


@jax.jit
def kernel(u, v, user_emb, item_emb):
    u2 = u.astype(jnp.int32).reshape(NUM_WORKERS, BPW)
    v2 = v.astype(jnp.int32).reshape(NUM_WORKERS, BPW)
    return _mf_kernel(u2, v2, user_emb, item_emb)
